# Initial kernel scaffold; baseline (speedup 1.0000x reference)
import jax
import jax.numpy as jnp
from jax.experimental import pallas as pl


def kernel(target, reco, in_pid, out_pid):
    raise NotImplementedError("write your pallas kernel here")


# speedup vs baseline: 2.8839x; 2.8839x over previous
import functools

import jax
import jax.numpy as jnp
from jax.experimental import pallas as pl
from jax.experimental.pallas import tpu as pltpu

_B, _N, _D = 16, 2048, 3
_TILE = 256
_BIG = 1e30


def _chamfer_tc_body(x_ref, yt_ref, inp_ref, outp_ref, acc_ref):
    b = pl.program_id(0)

    x = x_ref[0]
    in_pid = inp_ref[0]
    out_pid = outp_ref[0]

    in_mask_c = in_pid != 0
    out_mask_r = out_pid != 0
    zero_mask_r = jnp.logical_not(out_mask_r)

    n_in = jnp.sum(in_mask_c.astype(jnp.float32))
    n_out = jnp.sum(out_mask_r.astype(jnp.float32))
    n_zero = jnp.float32(_N) - n_out

    y0 = yt_ref[0, 0:1, :]
    y1 = yt_ref[0, 1:2, :]
    y2 = yt_ref[0, 2:3, :]

    x0c = x[:, 0:1]
    x1c = x[:, 1:2]
    x2c = x[:, 2:3]
    x_norm = jnp.sqrt(x0c * x0c + x1c * x1c + x2c * x2c)
    x_norm_sum = jnp.sum(jnp.where(in_mask_c, x_norm, 0.0))

    y_norm = jnp.sqrt(y0 * y0 + y1 * y1 + y2 * y2)
    y_zero_sum = jnp.sum(jnp.where(zero_mask_r, y_norm, 0.0))

    pen_out = jnp.where(out_mask_r, 0.0, _BIG)

    colmin = jnp.full((1, _N), _BIG, dtype=jnp.float32)
    rowsum = jnp.float32(0.0)
    for t in range(_N // _TILE):
        sl = slice(t * _TILE, (t + 1) * _TILE)
        xs0 = x0c[sl, :]
        xs1 = x1c[sl, :]
        xs2 = x2c[sl, :]
        d0 = xs0 - y0
        d1 = xs1 - y1
        d2 = xs2 - y2
        dsq = d0 * d0 + d1 * d1 + d2 * d2

        in_m_t = in_mask_c[sl, :]
        row_min = jnp.min(dsq + pen_out, axis=1, keepdims=True)
        rowsum += jnp.sum(jnp.where(in_m_t, jnp.sqrt(row_min), 0.0))

        pen_in_t = jnp.where(in_m_t, 0.0, _BIG)
        colmin = jnp.minimum(
            colmin, jnp.min(dsq + pen_in_t, axis=0, keepdims=True))

    sum_yx = jnp.sum(jnp.where(out_mask_r, jnp.sqrt(colmin), 0.0))

    n_in_part = jnp.maximum(1.0, n_in)
    n_out_part = jnp.maximum(1.0, n_out)
    n_zero_part = jnp.maximum(1.0, n_zero)

    chamfer = 0.5 * (rowsum / n_out_part + sum_yx / n_in_part)
    contrib = jnp.where(
        n_out == 0.0,
        x_norm_sum / n_in_part,
        jnp.where(n_in == 0.0, 0.0, chamfer),
    )
    ez = y_zero_sum / n_zero_part

    row_idx = jax.lax.broadcasted_iota(jnp.int32, (8, 128), 0)
    val = jnp.where(row_idx == 0, contrib, jnp.where(row_idx == 1, ez, 0.0))
    val = val * (1.0 / _B)

    @pl.when(b == 0)
    def _():
        acc_ref[...] = jnp.zeros_like(acc_ref)

    acc_ref[...] += val


@jax.jit
def kernel(target, reco, in_pid, out_pid):
    yt = jnp.transpose(reco, (0, 2, 1))
    in_c = in_pid.astype(jnp.int32)[..., None]
    out_r = out_pid.astype(jnp.int32)[:, None, :]

    acc = pl.pallas_call(
        _chamfer_tc_body,
        grid=(_B,),
        in_specs=[
            pl.BlockSpec((1, _N, _D), lambda b: (b, 0, 0)),
            pl.BlockSpec((1, _D, _N), lambda b: (b, 0, 0)),
            pl.BlockSpec((1, _N, 1), lambda b: (b, 0, 0)),
            pl.BlockSpec((1, 1, _N), lambda b: (b, 0, 0)),
        ],
        out_specs=pl.BlockSpec((8, 128), lambda b: (0, 0)),
        out_shape=jax.ShapeDtypeStruct((8, 128), jnp.float32),
    )(target, yt, in_c, out_r)

    return acc[0, 0], acc[1, 0]
